# baseline (speedup 1.0000x reference)
import functools

import jax
import jax.numpy as jnp
from jax import lax
from jax.experimental import pallas as pl
from jax.experimental.pallas import tpu as pltpu
from jax.experimental.pallas import tpu_sc as plsc

B, L, D, V = 64, 512, 768, 100000
N = B * L
NC, NS = 2, 16
NW = NC * NS
CHUNK = 64
TOK_W = N // NW
NCHUNK = TOK_W // CHUNK

BLK = 512
NBLK = N // BLK


def _sc_gather_word(src_w, word_table):
    mesh = plsc.VectorSubcoreMesh(core_axis_name="c", subcore_axis_name="s")

    @functools.partial(
        pl.kernel,
        out_type=jax.ShapeDtypeStruct((N, D), jnp.float32),
        mesh=mesh,
        scratch_types=[
            pltpu.VMEM((NCHUNK, CHUNK), jnp.int32),
            pltpu.VMEM((CHUNK, D), jnp.float32),
            pltpu.VMEM((CHUNK, D), jnp.float32),
            pltpu.SemaphoreType.DMA,
            pltpu.SemaphoreType.DMA,
        ],
    )
    def gather_kernel(src_hbm, tab_hbm, out_hbm, idx_v, buf0, buf1,
                      sem0, sem1):
        wid = lax.axis_index("s") * NC + lax.axis_index("c")
        base = wid * TOK_W
        pltpu.sync_copy(src_hbm.at[wid], idx_v)
        bufs = (buf0, buf1)
        sems = (sem0, sem1)

        pltpu.async_copy(tab_hbm.at[idx_v.at[0]], buf0, sem0)

        def body(j, _):
            slot = lax.rem(j, 2)
            nslot = lax.rem(j + 1, 2)

            @pl.when(j + 1 < NCHUNK)
            def _():
                def fire(s):
                    pltpu.async_copy(tab_hbm.at[idx_v.at[j + 1]], bufs[s], sems[s])
                lax.cond(nslot == 0, lambda: fire(0), lambda: fire(1))

            def drain(s):
                pltpu.make_async_copy(tab_hbm.at[idx_v.at[j]], bufs[s], sems[s]).wait()
                pltpu.sync_copy(bufs[s], out_hbm.at[pl.ds(base + j * CHUNK, CHUNK)])
            lax.cond(slot == 0, lambda: drain(0), lambda: drain(1))
            return 0

        lax.fori_loop(0, NCHUNK, body, 0)

    return gather_kernel(src_w, word_table)


def _tc_body(g_r, pos_r, seg_r, typ_r, ptab_r, sttab_r, gam_r, bet_r, out_r):
    posb = pos_r[0]
    segb = seg_r[0]
    typb = typ_r[0]
    p_iota = lax.broadcasted_iota(jnp.int32, (512, BLK), 0)
    oh_p = (p_iota == posb).astype(jnp.bfloat16)
    st_iota = lax.broadcasted_iota(jnp.int32, (32, BLK), 0)
    oh_st = ((st_iota == segb) | (st_iota == typb + 3)).astype(jnp.bfloat16)
    dn = (((0,), (0,)), ((), ()))
    small = (lax.dot_general(oh_p, ptab_r[...], dn,
                             preferred_element_type=jnp.float32)
             + lax.dot_general(oh_st, sttab_r[...], dn,
                               preferred_element_type=jnp.float32))
    x = g_r[...] + small
    mean = jnp.mean(x, axis=1, keepdims=True)
    ex2 = jnp.mean(x * x, axis=1, keepdims=True)
    r = lax.rsqrt(ex2 - mean * mean + 1e-6)
    a = r * gam_r[...]
    out_r[...] = x * a + (bet_r[...] - mean * a)


def kernel(ids, src, seg, type, concept_ent_pairs, edge_idx, pos, need_gnn,
           word_table, token_type_table, pos_table, seg_table, gamma, beta):
    src_w = src.reshape(NW, NCHUNK, CHUNK).astype(jnp.int32)
    ptab = pos_table.astype(jnp.bfloat16)
    sttab = jnp.concatenate(
        [seg_table, token_type_table, jnp.zeros((8, D), jnp.float32)],
        axis=0).astype(jnp.bfloat16)
    pos_i = pos.reshape(NBLK, 1, BLK).astype(jnp.int32)
    seg_i = seg.reshape(NBLK, 1, BLK).astype(jnp.int32)
    typ_i = type.reshape(NBLK, 1, BLK).astype(jnp.int32)
    gam = gamma.reshape(1, D)
    bet = beta.reshape(1, D)

    g = _sc_gather_word(src_w, word_table)

    id_spec = pl.BlockSpec((1, 1, BLK), lambda i: (i, 0, 0))
    out = pl.pallas_call(
        _tc_body,
        grid=(NBLK,),
        in_specs=[pl.BlockSpec((BLK, D), lambda i: (i, 0)),
                  id_spec, id_spec, id_spec,
                  pl.BlockSpec((512, D), lambda i: (0, 0)),
                  pl.BlockSpec((32, D), lambda i: (0, 0)),
                  pl.BlockSpec((1, D), lambda i: (0, 0)),
                  pl.BlockSpec((1, D), lambda i: (0, 0))],
        out_specs=pl.BlockSpec((BLK, D), lambda i: (i, 0)),
        out_shape=jax.ShapeDtypeStruct((N, D), jnp.float32),
    )(g, pos_i, seg_i, typ_i, ptab, sttab, gam, bet)
    return out.reshape(B, L, D)


# speedup vs baseline: 1.0250x; 1.0250x over previous
import functools

import jax
import jax.numpy as jnp
from jax import lax
from jax.experimental import pallas as pl
from jax.experimental.pallas import tpu as pltpu
from jax.experimental.pallas import tpu_sc as plsc

B, L, D, V = 64, 512, 768, 100000
N = B * L
NC, NS = 2, 16
NW = NC * NS
CHUNK = 64

S = 2
NSL = N // S
TOK_W = NSL // NW
NCHUNK = TOK_W // CHUNK

BLK = 512
BLKS_S = NSL // BLK


def _sc_gather_word(src_w, word_table, dep=None):
    mesh = plsc.VectorSubcoreMesh(core_axis_name="c", subcore_axis_name="s")

    def body(src_hbm, tab_hbm, out_hbm, idx_v, buf0, buf1, sem0, sem1):
        wid = lax.axis_index("s") * NC + lax.axis_index("c")
        base = wid * TOK_W
        pltpu.sync_copy(src_hbm.at[wid], idx_v)
        bufs = (buf0, buf1)
        sems = (sem0, sem1)

        pltpu.async_copy(tab_hbm.at[idx_v.at[0]], buf0, sem0)

        def step(j, _):
            slot = lax.rem(j, 2)
            nslot = lax.rem(j + 1, 2)

            @pl.when(j + 1 < NCHUNK)
            def _():
                def fire(s):
                    pltpu.async_copy(tab_hbm.at[idx_v.at[j + 1]], bufs[s], sems[s])
                lax.cond(nslot == 0, lambda: fire(0), lambda: fire(1))

            def drain(s):
                pltpu.make_async_copy(tab_hbm.at[idx_v.at[j]], bufs[s], sems[s]).wait()
                pltpu.sync_copy(bufs[s], out_hbm.at[pl.ds(base + j * CHUNK, CHUNK)])
            lax.cond(slot == 0, lambda: drain(0), lambda: drain(1))
            return 0

        lax.fori_loop(0, NCHUNK, step, 0)

    scratch = [
        pltpu.VMEM((NCHUNK, CHUNK), jnp.int32),
        pltpu.VMEM((CHUNK, D), jnp.float32),
        pltpu.VMEM((CHUNK, D), jnp.float32),
        pltpu.SemaphoreType.DMA,
        pltpu.SemaphoreType.DMA,
    ]
    out_type = jax.ShapeDtypeStruct((NSL, D), jnp.float32)

    if dep is None:
        @functools.partial(pl.kernel, out_type=out_type, mesh=mesh,
                           scratch_types=scratch)
        def gather_kernel(src_hbm, tab_hbm, out_hbm, *rest):
            body(src_hbm, tab_hbm, out_hbm, *rest)
        return gather_kernel(src_w, word_table)

    @functools.partial(pl.kernel, out_type=out_type, mesh=mesh,
                       scratch_types=scratch)
    def gather_kernel_dep(src_hbm, tab_hbm, dep_hbm, out_hbm, *rest):
        del dep_hbm
        body(src_hbm, tab_hbm, out_hbm, *rest)
    return gather_kernel_dep(src_w, word_table, dep)


def _emb_ln(g, posb, segb, typb, ptab, sttab, gam, bet):
    p_iota = lax.broadcasted_iota(jnp.int32, (512, BLK), 0)
    oh_p = (p_iota == posb).astype(jnp.bfloat16)
    st_iota = lax.broadcasted_iota(jnp.int32, (32, BLK), 0)
    oh_st = ((st_iota == segb) | (st_iota == typb + 3)).astype(jnp.bfloat16)
    dn = (((0,), (0,)), ((), ()))
    small = (lax.dot_general(oh_p, ptab, dn, preferred_element_type=jnp.float32)
             + lax.dot_general(oh_st, sttab, dn,
                               preferred_element_type=jnp.float32))
    x = g + small
    mean = jnp.mean(x, axis=1, keepdims=True)
    ex2 = jnp.mean(x * x, axis=1, keepdims=True)
    r = lax.rsqrt(ex2 - mean * mean + 1e-6)
    a = r * gam
    return x * a + (bet - mean * a)


def _tc_body0(g_r, pos_r, seg_r, typ_r, ptab_r, sttab_r, gam_r, bet_r, out_r):
    out_r[...] = _emb_ln(g_r[...], pos_r[0], seg_r[0], typ_r[0],
                         ptab_r[...], sttab_r[...], gam_r[...], bet_r[...])


def _tc_body1(buf_r, g_r, pos_r, seg_r, typ_r, ptab_r, sttab_r, gam_r,
              bet_r, out_r):
    del buf_r
    out_r[...] = _emb_ln(g_r[...], pos_r[0], seg_r[0], typ_r[0],
                         ptab_r[...], sttab_r[...], gam_r[...], bet_r[...])


_ID_SPEC = pl.BlockSpec((1, 1, BLK), lambda i: (i, 0, 0))
_TAB_SPECS = [
    pl.BlockSpec((512, D), lambda i: (0, 0)),
    pl.BlockSpec((32, D), lambda i: (0, 0)),
    pl.BlockSpec((1, D), lambda i: (0, 0)),
    pl.BlockSpec((1, D), lambda i: (0, 0)),
]


def kernel(ids, src, seg, type, concept_ent_pairs, edge_idx, pos, need_gnn,
           word_table, token_type_table, pos_table, seg_table, gamma, beta):
    src_w = src.reshape(S, NW, NCHUNK, CHUNK).astype(jnp.int32)
    ptab = pos_table.astype(jnp.bfloat16)
    sttab = jnp.concatenate(
        [seg_table, token_type_table, jnp.zeros((8, D), jnp.float32)],
        axis=0).astype(jnp.bfloat16)
    pos_i = pos.reshape(S, BLKS_S, 1, BLK).astype(jnp.int32)
    seg_i = seg.reshape(S, BLKS_S, 1, BLK).astype(jnp.int32)
    typ_i = type.reshape(S, BLKS_S, 1, BLK).astype(jnp.int32)
    gam = gamma.reshape(1, D)
    bet = beta.reshape(1, D)

    g0 = _sc_gather_word(src_w[0], word_table)
    g1 = _sc_gather_word(src_w[1], word_table, dep=g0)

    buf = pl.pallas_call(
        _tc_body0,
        grid=(BLKS_S,),
        in_specs=[pl.BlockSpec((BLK, D), lambda i: (i, 0)),
                  _ID_SPEC, _ID_SPEC, _ID_SPEC, *_TAB_SPECS],
        out_specs=pl.BlockSpec((BLK, D), lambda i: (i, 0)),
        out_shape=jax.ShapeDtypeStruct((N, D), jnp.float32),
    )(g0, pos_i[0], seg_i[0], typ_i[0], ptab, sttab, gam, bet)

    out = pl.pallas_call(
        _tc_body1,
        grid=(BLKS_S,),
        in_specs=[pl.BlockSpec(memory_space=pl.ANY),
                  pl.BlockSpec((BLK, D), lambda i: (i, 0)),
                  _ID_SPEC, _ID_SPEC, _ID_SPEC, *_TAB_SPECS],
        out_specs=pl.BlockSpec((BLK, D), lambda i: (BLKS_S + i, 0)),
        out_shape=jax.ShapeDtypeStruct((N, D), jnp.float32),
        input_output_aliases={0: 0},
    )(buf, g1, pos_i[1], seg_i[1], typ_i[1], ptab, sttab, gam, bet)
    return out.reshape(B, L, D)
